# baseline (speedup 1.0000x reference)
import functools

import jax
import jax.numpy as jnp
import numpy as np
from jax import lax
from jax.experimental import pallas as pl
from jax.experimental.pallas import tpu as pltpu
from jax.experimental.pallas import tpu_sc as plsc

B = 4096
L = 200
D = 64
NC = 2
NS = 16
NW = NC * NS
BW = B // NW
NLG = L // 8
NG = BW // 16
NBT = B // 128


def _pos_encoding():
    position = np.arange(0, L, dtype=np.float32)[:, None]
    div_term = np.exp(np.arange(0, D, 2, dtype=np.float32) * (-np.log(10000.0) / D))
    pe = np.zeros((L, D), dtype=np.float32)
    pe[:, 0::2] = np.sin(position * div_term)
    pe[:, 1::2] = np.cos(position * div_term)
    return jnp.asarray(pe)


_MESH = plsc.VectorSubcoreMesh(core_axis_name="c", subcore_axis_name="s")


@functools.partial(
    pl.kernel,
    out_type=jax.ShapeDtypeStruct((L, D // 8, NBT, 8, 128), jnp.float32),
    mesh=_MESH,
    scratch_types=[
        pltpu.VMEM((NLG, 8, BW), jnp.int32),
        pltpu.VMEM((L, D), jnp.float32),
        pltpu.VMEM((4, BW, D), jnp.float32),
        pltpu.VMEM((2, D, BW), jnp.float32),
        pltpu.SemaphoreType.DMA,
        pltpu.SemaphoreType.DMA,
        pltpu.SemaphoreType.DMA,
        pltpu.SemaphoreType.DMA,
        pltpu.SemaphoreType.DMA,
        pltpu.SemaphoreType.DMA,
    ],
    compiler_params=pltpu.CompilerParams(
        use_tc_tiling_on_sc=False, needs_layout_passes=False
    ),
)
def _embed(idx3, pe2, table, out5,
           idx_all, pe_v, rows_v, trans_v,
           semg0, semg1, semg2, semg3, semo0, semo1):
    w = lax.axis_index("s") * NC + lax.axis_index("c")
    bcol = w * BW
    semg = (semg0, semg1, semg2, semg3)
    semo = (semo0, semo1)
    iota = jnp.arange(16, dtype=jnp.int32)

    pltpu.sync_copy(idx3.at[:, :, pl.ds(bcol, BW)], idx_all)
    pltpu.sync_copy(pe2, pe_v)

    GSPLIT = 4
    GCH = BW // GSPLIT

    def fire_gather(lgx, rx, nb):
        for c in range(GSPLIT):
            pltpu.async_copy(
                table.at[idx_all.at[lgx, rx].at[pl.ds(GCH * c, GCH)]],
                rows_v.at[nb].at[pl.ds(GCH * c, GCH)],
                semg[nb],
            )

    def wait_gather(nb):
        for c in range(GSPLIT):
            pltpu.make_async_copy(
                table.at[idx_all.at[0, 0].at[pl.ds(GCH * c, GCH)]],
                rows_v.at[nb].at[pl.ds(GCH * c, GCH)],
                semg[nb],
            ).wait()

    def fire_store(l, b):
        for dt in range(D // 8):
            pltpu.async_copy(
                trans_v.at[b].at[pl.ds(8 * dt, 8)],
                out5.at[l, dt, w],
                semo[b],
            )

    def wait_store(b):
        for dt in range(D // 8):
            pltpu.make_async_copy(
                trans_v.at[b].at[pl.ds(8 * dt, 8)],
                out5.at[0, dt, w],
                semo[b],
            ).wait()

    def compute(l, b, gb):
        lvec = jnp.full((16,), 0, jnp.int32) + l
        rowvs = [iota + 16 * g for g in range(NG)]

        @plsc.parallel_loop(0, D, step=1, unroll=4)
        def dbody(d):
            dvec = jnp.full((16,), 0, jnp.int32) + d
            pvec = plsc.load_gather(pe_v, [lvec, dvec])
            for g in range(NG):
                vals = plsc.load_gather(rows_v.at[gb], [rowvs[g], dvec])
                trans_v[b, d, pl.ds(16 * g, 16)] = vals + pvec

    fire_gather(0, 0, 0)
    fire_gather(0, 1, 1)

    def lg_body(lg, carry):
        for r in range(8):
            l = 8 * lg + r
            b = r % 2
            gb = r % 4
            pb = (r + 2) % 4
            if r < 6:
                fire_gather(lg, r + 2, pb)
            else:
                @pl.when(lg < NLG - 1)
                def _():
                    fire_gather(lg + 1, r - 6, pb)
            wait_gather(gb)
            if r < 2:
                @pl.when(lg >= 1)
                def _():
                    wait_store(b)
            else:
                wait_store(b)
            compute(l, b, gb)
            fire_store(l, b)
        return carry

    lax.fori_loop(0, NLG, lg_body, 0)
    wait_store(0)
    wait_store(1)


def kernel(indices, table):
    idx3 = indices.T.reshape(NLG, 8, B).astype(jnp.int32)
    out5 = _embed(idx3, _pos_encoding(), table)
    out = jnp.transpose(out5, (2, 4, 0, 1, 3)).reshape(B, L, D)
    return out


# speedup vs baseline: 7.9298x; 7.9298x over previous
import functools

import jax
import jax.numpy as jnp
import numpy as np
from jax import lax
from jax.experimental import pallas as pl
from jax.experimental.pallas import tpu as pltpu
from jax.experimental.pallas import tpu_sc as plsc

B = 4096
L = 200
D = 64
NC = 2
NS = 16
NW = NC * NS
BW = B // NW
NLG = L // 8
NG = BW // 16
NBT = B // 128


def _pos_encoding():
    position = np.arange(0, L, dtype=np.float32)[:, None]
    div_term = np.exp(np.arange(0, D, 2, dtype=np.float32) * (-np.log(10000.0) / D))
    pe = np.zeros((L, D), dtype=np.float32)
    pe[:, 0::2] = np.sin(position * div_term)
    pe[:, 1::2] = np.cos(position * div_term)
    return jnp.asarray(pe)


_MESH = plsc.VectorSubcoreMesh(core_axis_name="c", subcore_axis_name="s")


@functools.partial(
    pl.kernel,
    out_type=jax.ShapeDtypeStruct((L, D // 8, NBT, 8, 128), jnp.float32),
    mesh=_MESH,
    scratch_types=[
        pltpu.VMEM((NLG, 8, BW), jnp.int32),
        pltpu.VMEM((L, D), jnp.float32),
        pltpu.VMEM((4, BW, D), jnp.float32),
        pltpu.VMEM((2, D, BW + 1), jnp.float32),
        pltpu.SemaphoreType.DMA,
        pltpu.SemaphoreType.DMA,
        pltpu.SemaphoreType.DMA,
        pltpu.SemaphoreType.DMA,
        pltpu.SemaphoreType.DMA,
        pltpu.SemaphoreType.DMA,
    ],
    compiler_params=pltpu.CompilerParams(
        use_tc_tiling_on_sc=False, needs_layout_passes=False
    ),
)
def _embed(idx3, pe2, table, out5,
           idx_all, pe_v, rows_v, trans_v,
           semg0, semg1, semg2, semg3, semo0, semo1):
    w = lax.axis_index("s") * NC + lax.axis_index("c")
    bcol = w * BW
    semg = (semg0, semg1, semg2, semg3)
    semo = (semo0, semo1)
    iota = jnp.arange(16, dtype=jnp.int32)

    pltpu.sync_copy(idx3.at[:, :, pl.ds(bcol, BW)], idx_all)
    pltpu.sync_copy(pe2, pe_v)

    GSPLIT = 4
    GCH = BW // GSPLIT

    def fire_gather(lgx, rx, nb):
        for c in range(GSPLIT):
            pltpu.async_copy(
                table.at[idx_all.at[lgx, rx].at[pl.ds(GCH * c, GCH)]],
                rows_v.at[nb].at[pl.ds(GCH * c, GCH)],
                semg[nb],
            )

    def wait_gather(nb):
        for c in range(GSPLIT):
            pltpu.make_async_copy(
                table.at[idx_all.at[0, 0].at[pl.ds(GCH * c, GCH)]],
                rows_v.at[nb].at[pl.ds(GCH * c, GCH)],
                semg[nb],
            ).wait()

    def fire_store(l, b):
        for dt in range(D // 8):
            pltpu.async_copy(
                trans_v.at[b].at[pl.ds(8 * dt, 8), pl.ds(0, BW)],
                out5.at[l, dt, w],
                semo[b],
            )

    def wait_store(b):
        for dt in range(D // 8):
            pltpu.make_async_copy(
                trans_v.at[b].at[pl.ds(8 * dt, 8), pl.ds(0, BW)],
                out5.at[0, dt, w],
                semo[b],
            ).wait()

    def compute(l, b, gb):
        pecs = [pe_v[l, pl.ds(16 * c, 16)] for c in range(D // 16)]
        dvecs = [iota + 16 * c for c in range(D // 16)]

        @plsc.parallel_loop(0, BW, step=1, unroll=4)
        def tbody(t):
            tv = jnp.full((16,), 0, jnp.int32) + t
            for c in range(D // 16):
                vals = plsc.load_gather(rows_v.at[gb], [tv, dvecs[c]])
                plsc.store_scatter(
                    trans_v.at[b], [dvecs[c], tv], vals + pecs[c]
                )

    fire_gather(0, 0, 0)
    fire_gather(0, 1, 1)

    def lg_body(lg, carry):
        for r in range(8):
            l = 8 * lg + r
            b = r % 2
            gb = r % 4
            pb = (r + 2) % 4
            if r < 6:
                fire_gather(lg, r + 2, pb)
            else:
                @pl.when(lg < NLG - 1)
                def _():
                    fire_gather(lg + 1, r - 6, pb)
            wait_gather(gb)
            if r < 2:
                @pl.when(lg >= 1)
                def _():
                    wait_store(b)
            else:
                wait_store(b)
            compute(l, b, gb)
            fire_store(l, b)
        return carry

    lax.fori_loop(0, NLG, lg_body, 0)
    wait_store(0)
    wait_store(1)


def kernel(indices, table):
    idx3 = indices.T.reshape(NLG, 8, B).astype(jnp.int32)
    out5 = _embed(idx3, _pos_encoding(), table)
    out = jnp.transpose(out5, (2, 4, 0, 1, 3)).reshape(B, L, D)
    return out
